# baseline (speedup 1.0000x reference)
import jax
import jax.numpy as jnp
from jax.experimental import pallas as pl
from jax.experimental.pallas import tpu as pltpu

P = 16
E = 16
C = 96
FF = 96
EPS = 1e-6


def _moe_row(x_ref, wr_ref, w1_ref, w2_ref, y_ref, xt_ref, ot_ref):
    wr = wr_ref[:]
    w = x_ref.shape[-1]
    wp = w // P

    s = jnp.zeros((C, w), dtype=jnp.float32)
    for r in range(P):
        xr = x_ref[0, :, r, :]
        ms = jnp.mean(xr * xr, axis=0, keepdims=True)
        z = xr * jax.lax.rsqrt(ms + EPS)
        xt_ref[r] = z.T.astype(jnp.bfloat16)
        s = s + z

    wi = jax.lax.broadcasted_iota(jnp.int32, (w, wp), 0)
    ji = jax.lax.broadcasted_iota(jnp.int32, (w, wp), 1)
    seg = jnp.where(wi // P == ji, 1.0, 0.0)
    pooled = jax.lax.dot_general(
        s, seg, (((1,), (0,)), ((), ())),
        preferred_element_type=jnp.float32)
    logits = jax.lax.dot_general(
        pooled, wr, (((0,), (0,)), ((), ())),
        preferred_element_type=jnp.float32)
    lmax = jnp.max(logits, axis=1, keepdims=True)
    lane = jax.lax.broadcasted_iota(jnp.int32, (wp, E), 1)
    idx = jnp.min(jnp.where(logits >= lmax, lane, E), axis=1,
                  keepdims=True)

    for j in range(wp):
        e = idx[j, 0]
        xp = xt_ref[:, j * P:(j + 1) * P, :].reshape(P * P, C)
        w1 = w1_ref[e]
        w2 = w2_ref[e]
        h = jax.lax.dot_general(
            xp, w1, (((1,), (0,)), ((), ())),
            preferred_element_type=jnp.float32)
        h = (h * jax.nn.sigmoid(h)).astype(jnp.bfloat16)
        o = jax.lax.dot_general(
            h, w2, (((1,), (0,)), ((), ())),
            preferred_element_type=jnp.float32)
        ot_ref[:, j * P:(j + 1) * P, :] = o.reshape(P, P, C)

    for r in range(P):
        y_ref[0, :, r, :] = x_ref[0, :, r, :] + ot_ref[r].T


def kernel(x, g, Wr, W1, W2):
    B, Cc, H, W = x.shape
    Hp = H // P

    wrg = g[:, None] * Wr
    w1g = (g[None, :, None] * W1).astype(jnp.bfloat16)
    w2b = W2.astype(jnp.bfloat16)

    y = pl.pallas_call(
        _moe_row,
        grid=(B, Hp),
        in_specs=[
            pl.BlockSpec((1, Cc, P, W), lambda b, i: (b, 0, i, 0)),
            pl.BlockSpec((Cc, E), lambda b, i: (0, 0)),
            pl.BlockSpec((E, Cc, FF), lambda b, i: (0, 0, 0)),
            pl.BlockSpec((E, FF, Cc), lambda b, i: (0, 0, 0)),
        ],
        out_specs=pl.BlockSpec((1, Cc, P, W), lambda b, i: (b, 0, i, 0)),
        out_shape=jax.ShapeDtypeStruct((B, Cc, H, W), x.dtype),
        scratch_shapes=[
            pltpu.VMEM((P, W, Cc), jnp.bfloat16),
            pltpu.VMEM((P, W, Cc), jnp.float32),
        ],
    )(x, wrg, w1g, w2b)

    return y
